# baseline (speedup 1.0000x reference)
import functools

import jax
import jax.numpy as jnp
from jax import lax
from jax.experimental import pallas as pl
from jax.experimental.pallas import tpu as pltpu
from jax.experimental.pallas import tpu_sc as plsc

BATCH = 16384
EMB = 16
NC = 2
NS = 16
L = 16
NW = NC * NS
BPW = BATCH // NW
CHUNK = 128
NCHUNK = BPW // CHUNK

_mesh = plsc.VectorSubcoreMesh(core_axis_name="c", subcore_axis_name="s")


@functools.partial(
    pl.kernel,
    out_type=jax.ShapeDtypeStruct((BATCH,), jnp.float32),
    mesh=_mesh,
    scratch_types=[
        pltpu.VMEM((NCHUNK, CHUNK), jnp.int32),
        pltpu.VMEM((NCHUNK, CHUNK), jnp.int32),
        pltpu.VMEM((BPW, EMB), jnp.float32),
        pltpu.VMEM((BPW, EMB), jnp.float32),
        pltpu.VMEM((BPW,), jnp.float32),
        pltpu.SemaphoreType.DMA,
    ],
    compiler_params=pltpu.CompilerParams(
        needs_layout_passes=False, use_tc_tiling_on_sc=False),
)
def _mf_sc(u_id_hbm, i_id_hbm, user_hbm, item_hbm, out_hbm,
           u_idx, i_idx, u_rows, i_rows, out_v, sem):
    wid = lax.axis_index("s") * NC + lax.axis_index("c")

    pltpu.sync_copy(u_id_hbm.at[pl.ds(wid * NCHUNK, NCHUNK)], u_idx)
    pltpu.sync_copy(i_id_hbm.at[pl.ds(wid * NCHUNK, NCHUNK)], i_idx)

    copies = []
    for j in range(NCHUNK):
        rows = pl.ds(j * CHUNK, CHUNK)
        copies.append(pltpu.async_copy(user_hbm.at[u_idx.at[j]], u_rows.at[rows], sem))
        copies.append(pltpu.async_copy(item_hbm.at[i_idx.at[j]], i_rows.at[rows], sem))
    for cp in copies:
        cp.wait()

    lanes = lax.iota(jnp.int32, L)

    def group_body(g, carry):
        row = g * L + lanes
        acc = jnp.zeros((L,), jnp.float32)
        for e in range(EMB):
            col = jnp.full((L,), e, jnp.int32)
            uu = plsc.load_gather(u_rows, [row, col])
            ii = plsc.load_gather(i_rows, [row, col])
            acc = acc + uu * ii
        out_v[pl.ds(g * L, L)] = acc
        return carry

    lax.fori_loop(0, BPW // L, group_body, 0)

    pltpu.sync_copy(out_v, out_hbm.at[pl.ds(wid * BPW, BPW)])


def kernel(u_id, i_id, user_table, item_table):
    u2 = u_id.astype(jnp.int32).reshape(NW * NCHUNK, CHUNK)
    i2 = i_id.astype(jnp.int32).reshape(NW * NCHUNK, CHUNK)
    return _mf_sc(u2, i2, user_table, item_table)


# speedup vs baseline: 1.4840x; 1.4840x over previous
import functools

import jax
import jax.numpy as jnp
from jax import lax
from jax.experimental import pallas as pl
from jax.experimental.pallas import tpu as pltpu
from jax.experimental.pallas import tpu_sc as plsc

BATCH = 16384
EMB = 16
NC = 2
NS = 16
L = 16
NW = NC * NS
BPW = BATCH // NW
HALF = BPW // 2

_mesh = plsc.VectorSubcoreMesh(core_axis_name="c", subcore_axis_name="s")


@functools.partial(
    pl.kernel,
    out_type=jax.ShapeDtypeStruct((BATCH,), jnp.float32),
    mesh=_mesh,
    scratch_types=[
        pltpu.VMEM((BPW,), jnp.int32),
        pltpu.VMEM((BPW,), jnp.int32),
        pltpu.VMEM((HALF, EMB), jnp.float32),
        pltpu.VMEM((HALF, EMB), jnp.float32),
        pltpu.VMEM((BPW,), jnp.float32),
        pltpu.SemaphoreType.DMA,
        pltpu.SemaphoreType.DMA,
    ],
    compiler_params=pltpu.CompilerParams(needs_layout_passes=False),
)
def _mf_sc(u_id_hbm, i_id_hbm, user_hbm, item_hbm, out_hbm,
           u_vm, i_vm, u_buf, i_buf, out_v, sem_u, sem_i):
    wid = lax.axis_index("s") * NC + lax.axis_index("c")
    base = wid * BPW

    pltpu.sync_copy(u_id_hbm.at[pl.ds(base, BPW)], u_vm)
    pltpu.sync_copy(i_id_hbm.at[pl.ds(base, BPW)], i_vm)

    lanes = lax.iota(jnp.int32, L)

    for h in range(2):
        def issue(c, carry):
            uv = u_vm[pl.ds(h * HALF + c * L, L)]
            iv = i_vm[pl.ds(h * HALF + c * L, L)]
            for j in range(L):
                pltpu.async_copy(
                    user_hbm.at[pl.ds(uv[j], 1)],
                    u_buf.at[pl.ds(c * L + j, 1)], sem_u)
                pltpu.async_copy(
                    item_hbm.at[pl.ds(iv[j], 1)],
                    i_buf.at[pl.ds(c * L + j, 1)], sem_i)
            return carry

        lax.fori_loop(0, HALF // L, issue, 0)

        def drain(b, carry):
            pltpu.make_async_copy(
                user_hbm.at[pl.ds(0, 1)], u_buf.at[pl.ds(b, 1)], sem_u).wait()
            pltpu.make_async_copy(
                item_hbm.at[pl.ds(0, 1)], i_buf.at[pl.ds(b, 1)], sem_i).wait()
            return carry

        lax.fori_loop(0, HALF, drain, 0)

        def group_body(g, carry):
            row = g * L + lanes
            acc = jnp.zeros((L,), jnp.float32)
            for e in range(EMB):
                col = jnp.full((L,), e, jnp.int32)
                uu = plsc.load_gather(u_buf, [row, col])
                ii = plsc.load_gather(i_buf, [row, col])
                acc = acc + uu * ii
            out_v[pl.ds(h * HALF + g * L, L)] = acc
            return carry

        lax.fori_loop(0, HALF // L, group_body, 0)

    pltpu.sync_copy(out_v, out_hbm.at[pl.ds(base, BPW)])


def kernel(u_id, i_id, user_table, item_table):
    return _mf_sc(u_id.astype(jnp.int32), i_id.astype(jnp.int32),
                  user_table, item_table)


# speedup vs baseline: 1.4867x; 1.0019x over previous
import functools

import jax
import jax.numpy as jnp
from jax import lax
from jax.experimental import pallas as pl
from jax.experimental.pallas import tpu as pltpu
from jax.experimental.pallas import tpu_sc as plsc

BATCH = 16384
EMB = 16
NC = 2
NS = 16
L = 16
NW = NC * NS
BPW = BATCH // NW
HALF = BPW // 2

_mesh = plsc.VectorSubcoreMesh(core_axis_name="c", subcore_axis_name="s")


@functools.partial(
    pl.kernel,
    out_type=jax.ShapeDtypeStruct((BATCH,), jnp.float32),
    mesh=_mesh,
    scratch_types=[
        pltpu.VMEM((BPW,), jnp.int32),
        pltpu.VMEM((BPW,), jnp.int32),
        pltpu.VMEM((HALF, EMB), jnp.float32),
        pltpu.VMEM((HALF, EMB), jnp.float32),
        pltpu.VMEM((BPW,), jnp.float32),
        pltpu.SemaphoreType.DMA,
        pltpu.SemaphoreType.DMA,
    ],
    compiler_params=pltpu.CompilerParams(
        needs_layout_passes=False,
        skip_device_barrier=True,
        disable_bounds_checks=True,
        disable_semaphore_checks=True,
    ),
)
def _mf_sc(u_id_hbm, i_id_hbm, user_hbm, item_hbm, out_hbm,
           u_vm, i_vm, u_buf, i_buf, out_v, sem_u, sem_i):
    wid = lax.axis_index("s") * NC + lax.axis_index("c")
    base = wid * BPW

    pltpu.sync_copy(u_id_hbm.at[pl.ds(base, BPW)], u_vm)
    pltpu.sync_copy(i_id_hbm.at[pl.ds(base, BPW)], i_vm)

    lanes = lax.iota(jnp.int32, L)

    for h in range(2):
        def issue(c, carry):
            uv = u_vm[pl.ds(h * HALF + c * L, L)]
            iv = i_vm[pl.ds(h * HALF + c * L, L)]
            for j in range(L):
                pltpu.async_copy(
                    user_hbm.at[pl.ds(uv[j], 1)],
                    u_buf.at[pl.ds(c * L + j, 1)], sem_u)
                pltpu.async_copy(
                    item_hbm.at[pl.ds(iv[j], 1)],
                    i_buf.at[pl.ds(c * L + j, 1)], sem_i)
            return carry

        lax.fori_loop(0, HALF // L, issue, 0)

        def drain(b, carry):
            pltpu.make_async_copy(
                user_hbm.at[pl.ds(0, 1)], u_buf.at[pl.ds(b, 1)], sem_u).wait()
            pltpu.make_async_copy(
                item_hbm.at[pl.ds(0, 1)], i_buf.at[pl.ds(b, 1)], sem_i).wait()
            return carry

        lax.fori_loop(0, HALF, drain, 0)

        def group_body(g, carry):
            row = g * L + lanes
            acc = jnp.zeros((L,), jnp.float32)
            for e in range(EMB):
                col = jnp.full((L,), e, jnp.int32)
                uu = plsc.load_gather(u_buf, [row, col])
                ii = plsc.load_gather(i_buf, [row, col])
                acc = acc + uu * ii
            out_v[pl.ds(h * HALF + g * L, L)] = acc
            return carry

        lax.fori_loop(0, HALF // L, group_body, 0)

    pltpu.sync_copy(out_v, out_hbm.at[pl.ds(base, BPW)])


def kernel(u_id, i_id, user_table, item_table):
    return _mf_sc(u_id.astype(jnp.int32), i_id.astype(jnp.int32),
                  user_table, item_table)
